# baseline (speedup 1.0000x reference)
import functools

import jax
import jax.numpy as jnp
from jax import lax
from jax.experimental import pallas as pl
from jax.experimental.pallas import tpu as pltpu
from jax.experimental.pallas import tpu_sc as plsc

NUM_EXPERTS = 64
NUM_K = 8
CAPACITY = 40 * 8
BATCH = 4
SEQ = 2048
HIDDEN = 4096
BLK_T = 1024
CH = 256
NTOK = BATCH * SEQ
NEG = -3.0e38


def _matmul_body(x_ref, wt_ref, logits_ref):
    logits_ref[0] = jnp.dot(x_ref[0], wt_ref[...],
                            preferred_element_type=jnp.float32)


def _tc_logits(hidden_states, wt):
    return pl.pallas_call(
        _matmul_body,
        grid=(BATCH, SEQ // BLK_T),
        in_specs=[
            pl.BlockSpec((1, BLK_T, HIDDEN), lambda b, s: (b, s, 0)),
            pl.BlockSpec((HIDDEN, NUM_EXPERTS), lambda b, s: (0, 0)),
        ],
        out_specs=pl.BlockSpec((1, BLK_T, NUM_EXPERTS), lambda b, s: (b, s, 0)),
        out_shape=jax.ShapeDtypeStruct((BATCH, SEQ, NUM_EXPERTS), jnp.float32),
    )(hidden_states, wt)


def _sc_route(logits_flat, interpret=False):
    mesh = plsc.VectorSubcoreMesh(core_axis_name="c", subcore_axis_name="s")

    @functools.partial(
        pl.kernel,
        mesh=mesh,
        out_type=[
            jax.ShapeDtypeStruct((NTOK * 16,), jnp.int32),
            jax.ShapeDtypeStruct((NTOK * NUM_EXPERTS,), jnp.int32),
            jax.ShapeDtypeStruct((NTOK * NUM_EXPERTS,), jnp.int32),
            jax.ShapeDtypeStruct((NTOK * 16,), jnp.float32),
        ],
        scratch_types=[
            pltpu.VMEM((CH * NUM_EXPERTS,), jnp.float32),
            pltpu.VMEM((CH * NUM_EXPERTS,), jnp.int32),
            pltpu.VMEM((CH * NUM_EXPERTS,), jnp.int32),
            pltpu.VMEM((CH * 16,), jnp.int32),
            pltpu.VMEM((CH * 16,), jnp.float32),
            pltpu.VMEM((NUM_EXPERTS,), jnp.int32),
            pltpu.VMEM((16 * NUM_EXPERTS,), jnp.int32),
            pltpu.VMEM_SHARED((16 * NUM_EXPERTS,), jnp.int32),
        ],
        compiler_params=pltpu.CompilerParams(needs_layout_passes=False),
        interpret=interpret,
    )
    def route(lg_hbm, idx_hbm, cnt_hbm, msk_hbm, tpv_hbm,
              lg_v, cnt_v, msk_v, idx_v, tpv_v, tot_v, pref_v, shared):
        lane = lax.iota(jnp.int32, 16)
        cid = lax.axis_index("c")
        sid = lax.axis_index("s")
        wid = cid * 16 + sid
        base = wid * CH * NUM_EXPERTS

        pltpu.sync_copy(lg_hbm.at[pl.ds(base, CH * NUM_EXPERTS)], lg_v)

        def tok1(t, tot):
            t0, t1, t2, t3 = tot
            off = t * NUM_EXPERTS
            w0 = lg_v[pl.ds(off, 16)]
            w1 = lg_v[pl.ds(off + 16, 16)]
            w2 = lg_v[pl.ds(off + 32, 16)]
            w3 = lg_v[pl.ds(off + 48, 16)]

            m = jnp.max(jnp.maximum(jnp.maximum(w0, w1),
                                    jnp.maximum(w2, w3)))
            sumexp = (jnp.sum(jnp.exp(w0 - m)) + jnp.sum(jnp.exp(w1 - m))
                      + jnp.sum(jnp.exp(w2 - m)) + jnp.sum(jnp.exp(w3 - m)))
            rcpv = 1.0 / jnp.broadcast_to(sumexp, (16,))

            c0 = jnp.zeros((16,), jnp.int32)
            c1 = jnp.zeros((16,), jnp.int32)
            c2 = jnp.zeros((16,), jnp.int32)
            c3 = jnp.zeros((16,), jnp.int32)
            idxv = jnp.zeros((16,), jnp.int32)
            tpvv = jnp.zeros((16,), jnp.float32)
            for j in range(NUM_K):
                s = jnp.max(jnp.maximum(jnp.maximum(w0, w1),
                                        jnp.maximum(w2, w3)))
                cand0 = jnp.where(w0 == s, lane, NUM_EXPERTS)
                cand1 = jnp.where(w1 == s, lane + 16, NUM_EXPERTS)
                cand2 = jnp.where(w2 == s, lane + 32, NUM_EXPERTS)
                cand3 = jnp.where(w3 == s, lane + 48, NUM_EXPERTS)
                ei = jnp.min(jnp.minimum(jnp.minimum(cand0, cand1),
                                         jnp.minimum(cand2, cand3)))
                h0 = lane == ei
                h1 = lane + 16 == ei
                h2 = lane + 32 == ei
                h3 = lane + 48 == ei
                w0 = jnp.where(h0, NEG, w0)
                w1 = jnp.where(h1, NEG, w1)
                w2 = jnp.where(h2, NEG, w2)
                w3 = jnp.where(h3, NEG, w3)
                c0 = c0 + h0.astype(jnp.int32)
                c1 = c1 + h1.astype(jnp.int32)
                c2 = c2 + h2.astype(jnp.int32)
                c3 = c3 + h3.astype(jnp.int32)
                jlane = lane == j
                idxv = jnp.where(jlane, ei, idxv)
                pv = jnp.exp(jnp.broadcast_to(s - m, (16,))) * rcpv
                tpvv = jnp.where(jlane, pv, tpvv)

            cnt_v[pl.ds(off, 16)] = c0
            cnt_v[pl.ds(off + 16, 16)] = c1
            cnt_v[pl.ds(off + 32, 16)] = c2
            cnt_v[pl.ds(off + 48, 16)] = c3
            idx_v[pl.ds(t * 16, 16)] = idxv
            tpv_v[pl.ds(t * 16, 16)] = tpvv
            return (t0 + c0, t1 + c1, t2 + c2, t3 + c3)

        z = jnp.zeros((16,), jnp.int32)
        tot = lax.fori_loop(0, CH, tok1, (z, z, z, z))
        tot_v[pl.ds(0, 16)] = tot[0]
        tot_v[pl.ds(16, 16)] = tot[1]
        tot_v[pl.ds(32, 16)] = tot[2]
        tot_v[pl.ds(48, 16)] = tot[3]

        pltpu.sync_copy(tot_v, shared.at[pl.ds(sid * NUM_EXPERTS, NUM_EXPERTS)])
        plsc.subcore_barrier()
        pltpu.sync_copy(shared, pref_v)

        bstart = (sid // 8) * 8
        a0 = jnp.zeros((16,), jnp.int32)
        a1 = jnp.zeros((16,), jnp.int32)
        a2 = jnp.zeros((16,), jnp.int32)
        a3 = jnp.zeros((16,), jnp.int32)
        for r in range(16):
            use = (r >= bstart) & (r < sid)
            a0 = a0 + jnp.where(use, pref_v[pl.ds(r * 64, 16)], 0)
            a1 = a1 + jnp.where(use, pref_v[pl.ds(r * 64 + 16, 16)], 0)
            a2 = a2 + jnp.where(use, pref_v[pl.ds(r * 64 + 32, 16)], 0)
            a3 = a3 + jnp.where(use, pref_v[pl.ds(r * 64 + 48, 16)], 0)

        def tok2(t, acc):
            a0, a1, a2, a3 = acc
            off = t * NUM_EXPERTS
            c0 = cnt_v[pl.ds(off, 16)]
            c1 = cnt_v[pl.ds(off + 16, 16)]
            c2 = cnt_v[pl.ds(off + 32, 16)]
            c3 = cnt_v[pl.ds(off + 48, 16)]
            a0 = a0 + c0
            a1 = a1 + c1
            a2 = a2 + c2
            a3 = a3 + c3
            k0 = (a0 <= CAPACITY).astype(jnp.int32)
            k1 = (a1 <= CAPACITY).astype(jnp.int32)
            k2 = (a2 <= CAPACITY).astype(jnp.int32)
            k3 = (a3 <= CAPACITY).astype(jnp.int32)
            msk_v[pl.ds(off, 16)] = k0
            msk_v[pl.ds(off + 16, 16)] = k1
            msk_v[pl.ds(off + 32, 16)] = k2
            msk_v[pl.ds(off + 48, 16)] = k3
            cnt_v[pl.ds(off, 16)] = c0 * k0
            cnt_v[pl.ds(off + 16, 16)] = c1 * k1
            cnt_v[pl.ds(off + 32, 16)] = c2 * k2
            cnt_v[pl.ds(off + 48, 16)] = c3 * k3
            return (a0, a1, a2, a3)

        lax.fori_loop(0, CH, tok2, (a0, a1, a2, a3))

        pltpu.sync_copy(cnt_v, cnt_hbm.at[pl.ds(base, CH * NUM_EXPERTS)])
        pltpu.sync_copy(msk_v, msk_hbm.at[pl.ds(base, CH * NUM_EXPERTS)])
        pltpu.sync_copy(idx_v, idx_hbm.at[pl.ds(wid * CH * 16, CH * 16)])
        pltpu.sync_copy(tpv_v, tpv_hbm.at[pl.ds(wid * CH * 16, CH * 16)])

    return route(logits_flat)


@jax.jit
def kernel(hidden_states, W):
    logits = _tc_logits(hidden_states, W.T)
    idx16, cnt, msk, tpv16 = _sc_route(logits.reshape(-1))
    idx = idx16.reshape(NTOK, 16)[:, :NUM_K].reshape(BATCH, SEQ, NUM_K)
    tpv = tpv16.reshape(NTOK, 16)[:, :NUM_K].reshape(BATCH, SEQ, NUM_K)
    cnt = cnt.reshape(BATCH, SEQ, NUM_EXPERTS)
    msk_b = (msk > 0).reshape(BATCH, SEQ, NUM_EXPERTS)
    return (idx, cnt, msk_b, tpv, logits)


# speedup vs baseline: 2.0930x; 2.0930x over previous
import jax
import jax.numpy as jnp
from jax import lax
from jax.experimental import pallas as pl
from jax.experimental.pallas import tpu as pltpu

NUM_EXPERTS = 64
NUM_K = 8
CAPACITY = 40 * 8
BATCH = 4
SEQ = 2048
HIDDEN = 4096
BLK_T = 1024


def _router_body(x_ref, wt_ref, idx_ref, cnt_ref, mask_ref, topv_ref,
                 logits_ref, carry_ref, triu_ref):
    b = pl.program_id(0)
    s = pl.program_id(1)
    T = BLK_T

    @pl.when((b == 0) & (s == 0))
    def _():
        rr = lax.broadcasted_iota(jnp.int32, (T, T), 0)
        cc = lax.broadcasted_iota(jnp.int32, (T, T), 1)
        triu_ref[...] = (rr <= cc).astype(jnp.float32)

    @pl.when(s == 0)
    def _():
        carry_ref[...] = jnp.zeros_like(carry_ref)

    x = x_ref[0]
    wt = wt_ref[...]
    logits = jnp.dot(x, wt, preferred_element_type=jnp.float32)
    logits_ref[0] = logits

    lt = logits.T
    m = jnp.max(lt, axis=0, keepdims=True)
    sumexp = jnp.sum(jnp.exp(lt - m), axis=0, keepdims=True)

    iota_e = lax.broadcasted_iota(jnp.int32, (NUM_EXPERTS, T), 0)
    work = lt
    counts = jnp.zeros((NUM_EXPERTS, T), jnp.float32)
    idx_rows = []
    val_rows = []
    for _ in range(NUM_K):
        v = jnp.max(work, axis=0, keepdims=True)
        hit = work == v
        idx = jnp.min(jnp.where(hit, iota_e, NUM_EXPERTS),
                      axis=0, keepdims=True)
        onehot = (iota_e == idx)
        counts += onehot.astype(jnp.float32)
        work = jnp.where(onehot, -jnp.inf, work)
        idx_rows.append(idx)
        val_rows.append(v)
    idx_ref[0] = jnp.concatenate(idx_rows, axis=0).T
    vlog = jnp.concatenate(val_rows, axis=0)
    topv_ref[0] = (jnp.exp(vlog - m) / sumexp).T

    prio = jnp.dot(counts, triu_ref[...],
                   preferred_element_type=jnp.float32)
    prio = prio + carry_ref[...]
    carry_ref[...] = prio[:, T - 1:T]

    keep = prio <= float(CAPACITY)
    keep_i = keep.astype(jnp.int32)
    mask_ref[0] = keep_i.T > 0
    cnt_ref[0] = counts.astype(jnp.int32).T * keep_i.T


@jax.jit
def kernel(hidden_states, W):
    wt = W.T
    nblk = SEQ // BLK_T
    grid = (BATCH, nblk)
    out_shapes = (
        jax.ShapeDtypeStruct((BATCH, SEQ, NUM_K), jnp.int32),
        jax.ShapeDtypeStruct((BATCH, SEQ, NUM_EXPERTS), jnp.int32),
        jax.ShapeDtypeStruct((BATCH, SEQ, NUM_EXPERTS), jnp.bool_),
        jax.ShapeDtypeStruct((BATCH, SEQ, NUM_K), jnp.float32),
        jax.ShapeDtypeStruct((BATCH, SEQ, NUM_EXPERTS), jnp.float32),
    )
    tok_spec = lambda lastdim: pl.BlockSpec(
        (1, BLK_T, lastdim), lambda b, s: (b, s, 0))
    out = pl.pallas_call(
        _router_body,
        grid=grid,
        in_specs=[
            pl.BlockSpec((1, BLK_T, HIDDEN), lambda b, s: (b, s, 0)),
            pl.BlockSpec((HIDDEN, NUM_EXPERTS), lambda b, s: (0, 0)),
        ],
        out_specs=(
            tok_spec(NUM_K),
            tok_spec(NUM_EXPERTS),
            tok_spec(NUM_EXPERTS),
            tok_spec(NUM_K),
            tok_spec(NUM_EXPERTS),
        ),
        out_shape=out_shapes,
        scratch_shapes=[
            pltpu.VMEM((NUM_EXPERTS, 1), jnp.float32),
            pltpu.VMEM((BLK_T, BLK_T), jnp.float32),
        ],
    )(hidden_states, wt)
    idx, cnt, mask, topv, logits = out
    return (idx, cnt, mask, topv, logits)
